# Initial kernel scaffold; baseline (speedup 1.0000x reference)
import jax
import jax.numpy as jnp
from jax.experimental import pallas as pl


def kernel(x):
    raise NotImplementedError("write your pallas kernel here")


# speedup vs baseline: 3.1004x; 3.1004x over previous
import jax
import jax.numpy as jnp
from jax.experimental import pallas as pl
from jax.experimental.pallas import tpu as pltpu

_RB = 512
_CB = 256


def _scan_block(x_ref, o_ref, carry_ref):
    j = pl.program_id(1)

    @pl.when(j == 0)
    def _():
        carry_ref[...] = jnp.zeros_like(carry_ref)

    x = x_ref[...]
    carry = carry_ref[...]
    rows = jax.lax.broadcasted_iota(jnp.int32, (_CB, _CB), 0)
    cols = jax.lax.broadcasted_iota(jnp.int32, (_CB, _CB), 1)
    u_strict = (rows < cols).astype(jnp.float32)
    excl = jnp.dot(x, u_strict, preferred_element_type=jnp.float32)
    o_ref[...] = excl + carry
    carry_ref[...] = carry + jnp.sum(x, axis=1, keepdims=True)


def kernel(x):
    n_rows, n_cols = x.shape
    grid = (n_rows // _RB, n_cols // _CB)
    return pl.pallas_call(
        _scan_block,
        grid=grid,
        in_specs=[pl.BlockSpec((_RB, _CB), lambda i, j: (i, j))],
        out_specs=pl.BlockSpec((_RB, _CB), lambda i, j: (i, j)),
        out_shape=jax.ShapeDtypeStruct(x.shape, x.dtype),
        scratch_shapes=[pltpu.VMEM((_RB, 1), jnp.float32)],
        compiler_params=pltpu.CompilerParams(
            dimension_semantics=("parallel", "arbitrary"),
        ),
    )(x)
